# Initial kernel scaffold; baseline (speedup 1.0000x reference)
import jax
import jax.numpy as jnp
from jax.experimental import pallas as pl


def kernel(field, probe_x, probe_y):
    raise NotImplementedError("write your pallas kernel here")


# speedup vs baseline: 1.0192x; 1.0192x over previous
import functools

import jax
import jax.numpy as jnp
from jax import lax
from jax.experimental import pallas as pl
from jax.experimental.pallas import tpu as pltpu
from jax.experimental.pallas import tpu_sc as plsc

_L = 16


@functools.partial(jax.jit, static_argnums=(3, 4, 5))
def _probe_gather(ftab, px, py, planes, p_count, w):
    info = plsc.get_sparse_core_info()
    nw = info.num_cores * info.num_subcores
    planes_per_w = planes // nw
    chunks = p_count // _L
    plane_sz = ftab.shape[0] // planes
    slab = planes_per_w * p_count

    mesh = plsc.VectorSubcoreMesh(core_axis_name="c", subcore_axis_name="s")

    @functools.partial(
        pl.kernel,
        out_type=jax.ShapeDtypeStruct((planes * p_count,), jnp.float32),
        mesh=mesh,
        scratch_types=[
            pltpu.VMEM((p_count,), jnp.int32),
            pltpu.VMEM((p_count,), jnp.int32),
            pltpu.VMEM((planes_per_w, p_count), jnp.int32),
            pltpu.VMEM((slab,), jnp.float32),
            pltpu.SemaphoreType.DMA,
        ],
    )
    def k(ftab_hbm, px_hbm, py_hbm, out_hbm, px_v, py_v, idx_v, vals_v, sem):
        wid = lax.axis_index("s") * info.num_cores + lax.axis_index("c")
        pltpu.sync_copy(px_hbm, px_v)
        pltpu.sync_copy(py_hbm, py_v)

        for c in range(chunks):
            sl = pl.ds(c * _L, _L)
            base16 = px_v[sl] * w + py_v[sl]
            for i in range(planes_per_w):
                idx_v[i, sl] = base16 + (wid * planes_per_w + i) * plane_sz

        copies = []
        for i in range(planes_per_w):
            copies.append(
                pltpu.make_async_copy(
                    ftab_hbm.at[idx_v.at[i]],
                    vals_v.at[pl.ds(i * p_count, p_count)],
                    sem,
                )
            )
            copies[-1].start()
        for cp in copies:
            cp.wait()

        pltpu.sync_copy(vals_v, out_hbm.at[pl.ds(wid * slab, slab)])

    return k(ftab, px, py)


def kernel(field, probe_x, probe_y):
    b, ch, h, w = field.shape
    p_count = probe_x.shape[0]
    planes = b * ch
    ftab = field.reshape(planes * h * w)
    px = probe_x.astype(jnp.int32)
    py = probe_y.astype(jnp.int32)
    out = _probe_gather(ftab, px, py, planes, p_count, w)
    return out.reshape(b, ch, p_count)


# speedup vs baseline: 4.0606x; 3.9841x over previous
import functools

import jax
import jax.numpy as jnp
from jax import lax
from jax.experimental import pallas as pl
from jax.experimental.pallas import tpu as pltpu
from jax.experimental.pallas import tpu_sc as plsc

_L = 16


@functools.partial(jax.jit, static_argnums=(3,))
def _probe_gather(ftab, px, py, planes):
    rows, w = ftab.shape
    h = rows // planes
    p_count = px.shape[0]
    info = plsc.get_sparse_core_info()
    nw = info.num_cores * info.num_subcores
    planes_per_w = planes // nw
    chunks = p_count // _L
    slab = planes_per_w * p_count

    mesh = plsc.VectorSubcoreMesh(core_axis_name="c", subcore_axis_name="s")

    @functools.partial(
        pl.kernel,
        out_type=jax.ShapeDtypeStruct((nw, slab), jnp.float32),
        mesh=mesh,
        compiler_params=pltpu.CompilerParams(needs_layout_passes=False),
        scratch_types=[
            pltpu.VMEM((p_count,), jnp.int32),
            pltpu.VMEM((p_count,), jnp.int32),
            pltpu.VMEM((planes_per_w, p_count), jnp.int32),
            pltpu.VMEM((p_count, 512), jnp.float32),
            pltpu.VMEM((slab,), jnp.float32),
            pltpu.SemaphoreType.DMA,
        ],
    )
    def k(ftab_hbm, px_hbm, py_hbm, out_hbm, px_v, py_v, idx_v, rows_v,
          out_v, sem):
        wid = lax.axis_index("s") * info.num_cores + lax.axis_index("c")
        pltpu.sync_copy(px_hbm, px_v)
        pltpu.sync_copy(py_hbm, py_v)

        for c in range(chunks):
            sl = pl.ds(c * _L, _L)
            x16 = px_v[sl]
            for i in range(planes_per_w):
                idx_v[i, sl] = x16 + (wid * planes_per_w + i) * h

        for i in range(planes_per_w):
            cp = pltpu.make_async_copy(ftab_hbm.at[idx_v.at[i]], rows_v, sem)
            cp.start()
            cp.wait()
            for c in range(chunks):
                sl = pl.ds(c * _L, _L)
                p_ids = lax.iota(jnp.int32, _L) + c * _L
                y16 = py_v[sl]
                out_v[pl.ds(i * p_count + c * _L, _L)] = plsc.load_gather(
                    rows_v, [p_ids, y16]
                )

        pltpu.sync_copy(out_v, out_hbm.at[wid])

    return k(ftab, px, py)


def kernel(field, probe_x, probe_y):
    b, ch, h, w = field.shape
    p_count = probe_x.shape[0]
    planes = b * ch
    ftab = field.reshape(planes * h, w)
    px = probe_x.astype(jnp.int32)
    py = probe_y.astype(jnp.int32)
    out = _probe_gather(ftab, px, py, planes)
    return out.reshape(b, ch, p_count)


# speedup vs baseline: 5.9107x; 1.4556x over previous
import functools

import jax
import jax.numpy as jnp
from jax import lax
from jax.experimental import pallas as pl
from jax.experimental.pallas import tpu as pltpu
from jax.experimental.pallas import tpu_sc as plsc

_L = 16
_B = 128


@functools.partial(jax.jit, static_argnums=(3,))
def _probe_gather(ftab, px, py, planes):
    rows, w = ftab.shape
    h = rows // planes
    p_count = px.shape[0]
    info = plsc.get_sparse_core_info()
    nw = info.num_cores * info.num_subcores
    ppw = p_count // nw
    pchunks = planes // _L
    half = planes // 2

    mesh = plsc.VectorSubcoreMesh(core_axis_name="c", subcore_axis_name="s")

    @functools.partial(
        pl.kernel,
        out_type=jax.ShapeDtypeStruct((p_count, planes), jnp.float32),
        mesh=mesh,
        compiler_params=pltpu.CompilerParams(needs_layout_passes=False),
        scratch_types=[
            pltpu.VMEM((p_count + _L,), jnp.int32),
            pltpu.VMEM((p_count + _L,), jnp.int32),
            pltpu.VMEM((ppw * 2, half), jnp.int32),
            pltpu.VMEM((2 * planes, _B), jnp.float32),
            pltpu.VMEM((ppw, planes), jnp.float32),
            pltpu.SemaphoreType.DMA,
            pltpu.SemaphoreType.DMA,
        ],
    )
    def k(ftab_hbm, px_hbm, py_hbm, out_hbm, px_v, py_v, ridx_v,
          gran_v, out_v, sem0, sem1):
        wid = lax.axis_index("s") * info.num_cores + lax.axis_index("c")
        base = wid * ppw
        pltpu.sync_copy(px_hbm, px_v.at[pl.ds(0, p_count)])
        pltpu.sync_copy(py_hbm, py_v.at[pl.ds(0, p_count)])

        lane = lax.iota(jnp.int32, _L)
        sems = [sem0, sem1]

        def fire(j):
            buf = j % 2
            sx = px_v[pl.ds(base + j, _L)][0]
            sy = py_v[pl.ds(base + j, _L)][0]
            ystart = pl.multiple_of((sy >> 7) << 7, _B)
            for c in range(pchunks):
                hb, cc = divmod(c, pchunks // 2)
                pv = lane + c * _L
                ridx_v[j * 2 + hb, pl.ds(cc * _L, _L)] = pv * h + sx
            cps = []
            for hb in range(2):
                cp = pltpu.make_async_copy(
                    ftab_hbm.at[ridx_v.at[j * 2 + hb], pl.ds(ystart, _B)],
                    gran_v.at[pl.ds((buf * 2 + hb) * half, half)],
                    sems[buf],
                )
                cp.start()
                cps.append(cp)
            return cps

        def select(j, cps):
            buf = j % 2
            for cp in cps:
                cp.wait()
            sy = py_v[pl.ds(base + j, _L)][0]
            off = jnp.full((_L,), sy & (_B - 1), jnp.int32)
            for c in range(pchunks):
                rbase = buf * planes + c * _L
                out_v[j, pl.ds(c * _L, _L)] = plsc.load_gather(
                    gran_v, [lane + rbase, off]
                )

        pending = fire(0)
        for j in range(ppw):
            nxt = fire(j + 1) if j + 1 < ppw else None
            select(j, pending)
            pending = nxt

        pltpu.sync_copy(out_v, out_hbm.at[pl.ds(base, ppw)])

    return k(ftab, px, py)


def kernel(field, probe_x, probe_y):
    b, ch, h, w = field.shape
    p_count = probe_x.shape[0]
    planes = b * ch
    ftab = field.reshape(planes * h, w)
    px = probe_x.astype(jnp.int32)
    py = probe_y.astype(jnp.int32)
    out = _probe_gather(ftab, px, py, planes)
    return out.T.reshape(b, ch, p_count)


# speedup vs baseline: 5.9955x; 1.0143x over previous
import functools

import jax
import jax.numpy as jnp
from jax import lax
from jax.experimental import pallas as pl
from jax.experimental.pallas import tpu as pltpu
from jax.experimental.pallas import tpu_sc as plsc

_L = 16
_B = 128


@functools.partial(jax.jit, static_argnums=(3,))
def _probe_gather(ftab, px, py, planes):
    rows, w = ftab.shape
    h = rows // planes
    p_count = px.shape[0]
    info = plsc.get_sparse_core_info()
    nw = info.num_cores * info.num_subcores
    ppw = p_count // nw
    pchunks = planes // _L
    half = planes // 2

    mesh = plsc.VectorSubcoreMesh(core_axis_name="c", subcore_axis_name="s")

    @functools.partial(
        pl.kernel,
        out_type=jax.ShapeDtypeStruct((p_count, planes), jnp.float32),
        mesh=mesh,
        compiler_params=pltpu.CompilerParams(needs_layout_passes=False),
        scratch_types=[
            pltpu.VMEM((p_count + _L,), jnp.int32),
            pltpu.VMEM((p_count + _L,), jnp.int32),
            pltpu.VMEM((ppw * 2, half), jnp.int32),
            pltpu.VMEM((2 * planes, _B), jnp.float32),
            pltpu.VMEM((ppw, planes), jnp.float32),
            pltpu.SemaphoreType.DMA,
            pltpu.SemaphoreType.DMA,
        ],
    )
    def k(ftab_hbm, px_hbm, py_hbm, out_hbm, px_v, py_v, ridx_v,
          gran_v, out_v, sem0, sem1):
        wid = lax.axis_index("s") * info.num_cores + lax.axis_index("c")
        base = wid * ppw
        pltpu.sync_copy(px_hbm, px_v.at[pl.ds(0, p_count)])
        pltpu.sync_copy(py_hbm, py_v.at[pl.ds(0, p_count)])

        lane = lax.iota(jnp.int32, _L)
        sems = [sem0, sem1]

        def fire(j):
            buf = j % 2
            sx = px_v[pl.ds(base + j, _L)][0]
            sy = py_v[pl.ds(base + j, _L)][0]
            ystart = pl.multiple_of((sy >> 7) << 7, _B)
            hc = pchunks // 2

            def idx_body(c, _, j=j, sx=sx):
                pv = lane + c * _L
                ridx_v[j * 2 + c // hc, pl.ds((c % hc) * _L, _L)] = pv * h + sx
                return 0

            lax.fori_loop(0, pchunks, idx_body, 0)
            cps = []
            for hb in range(2):
                cp = pltpu.make_async_copy(
                    ftab_hbm.at[ridx_v.at[j * 2 + hb], pl.ds(ystart, _B)],
                    gran_v.at[pl.ds((buf * 2 + hb) * half, half)],
                    sems[buf],
                )
                cp.start()
                cps.append(cp)
            return cps

        def select(j, cps):
            buf = j % 2
            for cp in cps:
                cp.wait()
            sy = py_v[pl.ds(base + j, _L)][0]
            off = jnp.full((_L,), sy & (_B - 1), jnp.int32)

            def sel_body(c, _, j=j, buf=buf, off=off):
                rbase = buf * planes + c * _L
                out_v[j, pl.ds(c * _L, _L)] = plsc.load_gather(
                    gran_v, [lane + rbase, off]
                )
                return 0

            lax.fori_loop(0, pchunks, sel_body, 0)

        pending = fire(0)
        for j in range(ppw):
            nxt = fire(j + 1) if j + 1 < ppw else None
            select(j, pending)
            pending = nxt

        pltpu.sync_copy(out_v, out_hbm.at[pl.ds(base, ppw)])

    return k(ftab, px, py)


def kernel(field, probe_x, probe_y):
    b, ch, h, w = field.shape
    p_count = probe_x.shape[0]
    planes = b * ch
    ftab = field.reshape(planes * h, w)
    px = probe_x.astype(jnp.int32)
    py = probe_y.astype(jnp.int32)
    out = _probe_gather(ftab, px, py, planes)
    return out.T.reshape(b, ch, p_count)
